# baseline (speedup 1.0000x reference)
import functools

import jax
import jax.numpy as jnp
from jax import lax
from jax.experimental import pallas as pl
from jax.experimental.pallas import tpu as pltpu
from jax.experimental.pallas import tpu_sc as plsc

_INFO = plsc.get_sparse_core_info()
_NC = _INFO.num_cores
_NS = _INFO.num_subcores
_NW = _NC * _NS

_BB = 128
_NB = 3
_SPLIT = 8


@functools.lru_cache(maxsize=None)
def _make_gather(S, B, V, D, b_base, b_len):
    NP = S // 2
    D2 = 2 * D
    per_w = b_len // _NW
    mesh = plsc.VectorSubcoreMesh(core_axis_name="c", subcore_axis_name="s")

    @functools.partial(
        pl.kernel,
        mesh=mesh,
        out_type=jax.ShapeDtypeStruct((NP, b_len, D2), jnp.float32),
        compiler_params=pltpu.CompilerParams(use_tc_tiling_on_sc=False),
        scratch_types=[
            pltpu.VMEM((S, per_w), jnp.int32),
            pltpu.VMEM((_NB, 2, per_w, D), jnp.float32),
            pltpu.SemaphoreType.DMA,
            pltpu.SemaphoreType.DMA,
        ],
    )
    def gather_kernel(ids_hbm, table_hbm, out_hbm, idx_v, rows_v, gsem, osem):
        wid = lax.axis_index("s") * _NC + lax.axis_index("c")

        def chunk(b0):
            pltpu.sync_copy(ids_hbm.at[:, pl.ds(b_base + b0, per_w)], idx_v)

            def fire(p, buf):
                for s01 in range(2):
                    pltpu.async_copy(
                        table_hbm.at[idx_v.at[2 * p + s01]],
                        rows_v.at[buf].at[s01],
                        gsem,
                    )

            def drain(p, buf):
                for s01 in range(2):
                    pltpu.make_async_copy(
                        table_hbm.at[idx_v.at[2 * p + s01]],
                        rows_v.at[buf].at[s01],
                        gsem,
                    ).wait()

            def out_copies(p, buf):
                return [
                    pltpu.make_async_copy(
                        rows_v.at[buf].at[s01],
                        out_hbm.at[p, pl.ds(b0, per_w), pl.ds(s01 * D, D)],
                        osem,
                    )
                    for s01 in range(2)
                ]

            fire(0, 0)

            def body(p, carry):
                buf = lax.rem(p, _NB)
                nbuf = lax.rem(p + 1, _NB)

                @pl.when(p >= _NB - 1)
                def _():
                    for c in out_copies(p, nbuf):
                        c.wait()

                @pl.when(p + 1 < NP)
                def _():
                    fire(p + 1, nbuf)

                drain(p, buf)
                for c in out_copies(p, buf):
                    c.start()
                return carry

            lax.fori_loop(0, NP, body, 0)
            for t in range(_NB - 1):
                for c in out_copies(NP - 1 - t, lax.rem(NP - 1 - t, _NB)):
                    c.wait()

        chunk(wid * per_w)

    return gather_kernel


@functools.lru_cache(maxsize=None)
def _make_transpose(S, B, D, b_base, b_len, first):
    NP = S // 2
    D2 = 2 * D
    blk0 = b_base // _BB

    def body(*refs):
        x_ref, o_ref = refs[-2], refs[-1]
        x = x_ref[...]
        y = jnp.transpose(x, (0, 2, 1))
        o_ref[...] = y.reshape(S, D, _BB)

    piece_spec = pl.BlockSpec((NP, _BB, D2), lambda i: (0, i, 0))
    if first:
        in_specs = [piece_spec]
        aliases = {}
    else:
        in_specs = [pl.BlockSpec(memory_space=pl.ANY), piece_spec]
        aliases = {0: 0}

    return pl.pallas_call(
        body,
        grid=(b_len // _BB,),
        in_specs=in_specs,
        out_specs=pl.BlockSpec((S, D, _BB), lambda i: (0, 0, blk0 + i)),
        out_shape=jax.ShapeDtypeStruct((S, D, B), jnp.float32),
        input_output_aliases=aliases,
    )


def kernel(intent_ids, table):
    Bt, S = intent_ids.shape
    V, D = table.shape
    ids_t = jnp.transpose(intent_ids).astype(jnp.int32)
    piece = Bt // _SPLIT
    outs1 = [
        _make_gather(S, Bt, V, D, j * piece, piece)(ids_t, table)
        for j in range(_SPLIT)
    ]
    out2 = _make_transpose(S, Bt, D, 0, piece, True)(outs1[0])
    for j in range(1, _SPLIT):
        out2 = _make_transpose(S, Bt, D, j * piece, piece, False)(out2, outs1[j])
    return jnp.transpose(out2, (2, 0, 1))


# speedup vs baseline: 1.0046x; 1.0046x over previous
import functools

import jax
import jax.numpy as jnp
from jax import lax
from jax.experimental import pallas as pl
from jax.experimental.pallas import tpu as pltpu
from jax.experimental.pallas import tpu_sc as plsc

_INFO = plsc.get_sparse_core_info()
_NC = _INFO.num_cores
_NS = _INFO.num_subcores
_NW = _NC * _NS

_BB = 128
_NB = 3
_SPLIT = 4


@functools.lru_cache(maxsize=None)
def _make_gather(S, B, V, D, b_base, b_len):
    NP = S // 2
    D2 = 2 * D
    per_w = b_len // _NW
    mesh = plsc.VectorSubcoreMesh(core_axis_name="c", subcore_axis_name="s")

    @functools.partial(
        pl.kernel,
        mesh=mesh,
        out_type=jax.ShapeDtypeStruct((NP, b_len, D2), jnp.float32),
        compiler_params=pltpu.CompilerParams(use_tc_tiling_on_sc=False),
        scratch_types=[
            pltpu.VMEM((S, per_w), jnp.int32),
            pltpu.VMEM((_NB, 2, per_w, D), jnp.float32),
            pltpu.SemaphoreType.DMA,
            pltpu.SemaphoreType.DMA,
        ],
    )
    def gather_kernel(ids_hbm, table_hbm, out_hbm, idx_v, rows_v, gsem, osem):
        wid = lax.axis_index("s") * _NC + lax.axis_index("c")

        def chunk(b0):
            pltpu.sync_copy(ids_hbm.at[:, pl.ds(b_base + b0, per_w)], idx_v)

            def fire(p, buf):
                for s01 in range(2):
                    pltpu.async_copy(
                        table_hbm.at[idx_v.at[2 * p + s01]],
                        rows_v.at[buf].at[s01],
                        gsem,
                    )

            def drain(p, buf):
                for s01 in range(2):
                    pltpu.make_async_copy(
                        table_hbm.at[idx_v.at[2 * p + s01]],
                        rows_v.at[buf].at[s01],
                        gsem,
                    ).wait()

            def out_copies(p, buf):
                return [
                    pltpu.make_async_copy(
                        rows_v.at[buf].at[s01],
                        out_hbm.at[p, pl.ds(b0, per_w), pl.ds(s01 * D, D)],
                        osem,
                    )
                    for s01 in range(2)
                ]

            fire(0, 0)

            def body(p, carry):
                buf = lax.rem(p, _NB)
                nbuf = lax.rem(p + 1, _NB)

                @pl.when(p >= _NB - 1)
                def _():
                    for c in out_copies(p, nbuf):
                        c.wait()

                @pl.when(p + 1 < NP)
                def _():
                    fire(p + 1, nbuf)

                drain(p, buf)
                for c in out_copies(p, buf):
                    c.start()
                return carry

            lax.fori_loop(0, NP, body, 0)
            for t in range(_NB - 1):
                for c in out_copies(NP - 1 - t, lax.rem(NP - 1 - t, _NB)):
                    c.wait()

        chunk(wid * per_w)

    return gather_kernel


@functools.lru_cache(maxsize=None)
def _make_transpose(S, B, D, b_base, b_len, first):
    NP = S // 2
    D2 = 2 * D
    blk0 = b_base // _BB

    def body(*refs):
        x_ref, o_ref = refs[-2], refs[-1]
        x = x_ref[...]
        y = jnp.transpose(x, (0, 2, 1))
        o_ref[...] = y.reshape(S, D, _BB)

    piece_spec = pl.BlockSpec((NP, _BB, D2), lambda i: (0, i, 0))
    if first:
        in_specs = [piece_spec]
        aliases = {}
    else:
        in_specs = [pl.BlockSpec(memory_space=pl.ANY), piece_spec]
        aliases = {0: 0}

    return pl.pallas_call(
        body,
        grid=(b_len // _BB,),
        in_specs=in_specs,
        out_specs=pl.BlockSpec((S, D, _BB), lambda i: (0, 0, blk0 + i)),
        out_shape=jax.ShapeDtypeStruct((S, D, B), jnp.float32),
        input_output_aliases=aliases,
    )


def kernel(intent_ids, table):
    Bt, S = intent_ids.shape
    V, D = table.shape
    ids_t = jnp.transpose(intent_ids).astype(jnp.int32)
    piece = Bt // _SPLIT
    outs1 = [
        _make_gather(S, Bt, V, D, j * piece, piece)(ids_t, table)
        for j in range(_SPLIT)
    ]
    out2 = _make_transpose(S, Bt, D, 0, piece, True)(outs1[0])
    for j in range(1, _SPLIT):
        out2 = _make_transpose(S, Bt, D, j * piece, piece, False)(out2, outs1[j])
    return jnp.transpose(out2, (2, 0, 1))
